# baseline (speedup 1.0000x reference)
import functools

import jax
import jax.numpy as jnp
from jax import lax
from jax.experimental import pallas as pl
from jax.experimental.pallas import tpu as pltpu
from jax.experimental.pallas import tpu_sc as plsc

_NUM_WORKERS = 32


def _make_gather(idx_shape):
    n_rows, n_cols = idx_shape
    rows_w = n_rows // _NUM_WORKERS
    assert rows_w * _NUM_WORKERS == n_rows

    mesh = plsc.VectorSubcoreMesh(core_axis_name="c", subcore_axis_name="s")

    @functools.partial(
        pl.kernel,
        out_type=jax.ShapeDtypeStruct(idx_shape, jnp.float32),
        mesh=mesh,
        scratch_types=[
            pltpu.VMEM((rows_w, n_cols), jnp.int32),
            pltpu.VMEM((rows_w, n_cols), jnp.float32),
            pltpu.SemaphoreType.DMA,
        ],
        compiler_params=pltpu.CompilerParams(use_tc_tiling_on_sc=True),
    )
    def gather_kernel(table_hbm, idx_hbm, out_hbm, idx_v, val_v, sem):
        wid = lax.axis_index("s") * 2 + lax.axis_index("c")
        base = wid * rows_w
        pltpu.sync_copy(idx_hbm.at[pl.ds(base, rows_w), :], idx_v)

        unroll = 8

        def fire(j, carry):
            for g in range(unroll):
                r = j * unroll + g
                pltpu.async_copy(table_hbm.at[idx_v.at[r]], val_v.at[r], sem)
            return carry

        lax.fori_loop(0, rows_w // unroll, fire, 0)

        def drain(j, carry):
            for g in range(unroll):
                r = j * unroll + g
                pltpu.make_async_copy(
                    table_hbm.at[idx_v.at[r]], val_v.at[r], sem
                ).wait()
            return carry

        lax.fori_loop(0, rows_w // unroll, drain, 0)
        pltpu.sync_copy(val_v, out_hbm.at[pl.ds(base, rows_w), :])

    return gather_kernel


def kernel(fit_X_col, donors_idx):
    idx = donors_idx.astype(jnp.int32)
    return _make_gather(idx.shape)(fit_X_col, idx)


# speedup vs baseline: 1.0018x; 1.0018x over previous
import functools

import jax
import jax.numpy as jnp
from jax import lax
from jax.experimental import pallas as pl
from jax.experimental.pallas import tpu as pltpu
from jax.experimental.pallas import tpu_sc as plsc

_NUM_WORKERS = 32


def _make_gather(idx_shape):
    n_rows, n_cols = idx_shape
    rows_w = n_rows // _NUM_WORKERS
    assert rows_w * _NUM_WORKERS == n_rows

    mesh = plsc.VectorSubcoreMesh(core_axis_name="c", subcore_axis_name="s")

    @functools.partial(
        pl.kernel,
        out_type=jax.ShapeDtypeStruct(idx_shape, jnp.float32),
        mesh=mesh,
        scratch_types=[
            pltpu.VMEM((rows_w, n_cols), jnp.int32),
            pltpu.VMEM((rows_w, n_cols), jnp.float32),
            pltpu.SemaphoreType.DMA,
        ],
        compiler_params=pltpu.CompilerParams(needs_layout_passes=False),
    )
    def gather_kernel(table_hbm, idx_hbm, out_hbm, idx_v, val_v, sem):
        wid = lax.axis_index("s") * 2 + lax.axis_index("c")
        base = wid * rows_w
        pltpu.sync_copy(idx_hbm.at[pl.ds(base, rows_w), :], idx_v)

        unroll = 8

        def fire(j, carry):
            for g in range(unroll):
                r = j * unroll + g
                pltpu.async_copy(table_hbm.at[idx_v.at[r]], val_v.at[r], sem)
            return carry

        lax.fori_loop(0, rows_w // unroll, fire, 0)

        def drain(j, carry):
            for g in range(unroll):
                r = j * unroll + g
                pltpu.make_async_copy(
                    table_hbm.at[idx_v.at[r]], val_v.at[r], sem
                ).wait()
            return carry

        lax.fori_loop(0, rows_w // unroll, drain, 0)
        pltpu.sync_copy(val_v, out_hbm.at[pl.ds(base, rows_w), :])

    return gather_kernel


def kernel(fit_X_col, donors_idx):
    idx = donors_idx.astype(jnp.int32)
    return _make_gather(idx.shape)(fit_X_col, idx)
